# baseline (speedup 1.0000x reference)
import jax
import jax.numpy as jnp
from jax.experimental import pallas as pl
from jax.experimental.pallas import tpu as pltpu

_N = 15000
_D0 = 200
_BQ = 512
_BK = 512
_G = 30
_NP = _G * _BQ
_D = 256
_CA = 200
_CB = 201

_NEG = -jnp.inf
_PADV = -1e30
_MASKV = 1e32
_BIG = 2 ** 30


def _normalize_pad(a, bias_col_self, bias_col_other):
    an = a / jnp.maximum(jnp.linalg.norm(a, axis=-1, keepdims=True), 1e-8)
    ap = jnp.pad(an, ((0, _NP - _N), (0, _D - _D0)))
    inv = jnp.where(jnp.arange(_NP) >= _N, jnp.float32(_PADV), jnp.float32(0.0))
    ap = ap.at[:, bias_col_self].set(inv)
    ap = ap.at[:, bias_col_other].set(1.0)
    return ap


def _merge_top2(v1, i1, v2, i2, cand_v, cand_i):
    gt1 = cand_v > v1
    gt2 = cand_v > v2
    nv2 = jnp.where(gt1, v1, jnp.where(gt2, cand_v, v2))
    ni2 = jnp.where(gt1, i1, jnp.where(gt2, cand_i, i2))
    nv1 = jnp.where(gt1, cand_v, v1)
    ni1 = jnp.where(gt1, cand_i, i1)
    return nv1, ni1, nv2, ni2


def _block_top2_lanes(s, idx):
    m1 = jnp.max(s, axis=1, keepdims=True)
    a1 = jnp.min(jnp.where(s == m1, idx, _BIG), axis=1, keepdims=True)
    s2 = jnp.where(idx == a1, _NEG, s)
    m2 = jnp.max(s2, axis=1, keepdims=True)
    a2 = jnp.min(jnp.where(s2 == m2, idx, _BIG), axis=1, keepdims=True)
    return m1, a1, m2, a2


def _block_top2_sublanes(s, idx):
    m1 = jnp.max(s, axis=0, keepdims=True)
    a1 = jnp.min(jnp.where(s == m1, idx, _BIG), axis=0, keepdims=True)
    s2 = jnp.where(idx == a1, _NEG, s)
    m2 = jnp.max(s2, axis=0, keepdims=True)
    a2 = jnp.min(jnp.where(s2 == m2, idx, _BIG), axis=0, keepdims=True)
    return m1, a1, m2, a2


def _idx_dot(weights, mask, axis):
    return jax.lax.dot_general(
        weights, mask, (((1,), (axis,)), ((), ())),
        preferred_element_type=jnp.float32,
        precision=jax.lax.Precision.HIGHEST)


def _topk_kernel(xn_ref, yn_ref, xv_ref, xi_ref, yv_ref, yi_ref,
                 sf_ref, si_ref):
    q = pl.program_id(0)
    k = pl.program_id(1)

    @pl.when(jnp.logical_and(q == 0, k == 0))
    def _init():
        xv_ref[...] = jnp.full((2, _NP), _NEG, jnp.float32)
        xi_ref[...] = jnp.zeros((2, _NP), jnp.int32)
        yv_ref[...] = jnp.full((2, _NP), _NEG, jnp.float32)
        yi_ref[...] = jnp.zeros((2, _NP), jnp.int32)

    x = xn_ref[...]
    y = yn_ref[...]
    s = jax.lax.dot_general(
        x, y, (((1,), (1,)), ((), ())),
        preferred_element_type=jnp.float32,
        precision=jax.lax.Precision.DEFAULT)

    w = jnp.concatenate(
        [jax.lax.broadcasted_iota(jnp.int32, (1, _BK), 1).astype(jnp.float32),
         jnp.ones((1, _BK), jnp.float32)], axis=0)
    lane_q = jax.lax.broadcasted_iota(jnp.int32, (1, _BQ), 1)
    row_ok = (q * _BQ + lane_q) < _N
    col_ok = (k * _BK + lane_q) < _N

    m1 = jnp.max(s, axis=1, keepdims=True)
    eq1 = s == m1
    mk1 = jnp.where(eq1, 1.0, 0.0)
    d1 = _idx_dot(w, mk1, 1)
    s2 = jnp.where(eq1, s - _MASKV, s)
    m2 = jnp.max(s2, axis=1, keepdims=True)
    eq2 = s2 == m2
    mk2 = jnp.where(eq2, 1.0, 0.0)
    d2 = _idx_dot(w, mk2, 1)

    sf_ref[0:1, :] = jnp.transpose(m1)
    sf_ref[1:2, :] = jnp.transpose(m2)
    si_ref[0:1, :] = d1[0:1, :].astype(jnp.int32) + k * _BK
    si_ref[1:2, :] = d2[0:1, :].astype(jnp.int32) + k * _BK

    c1 = jnp.max(s, axis=0, keepdims=True)
    eqc1 = s == c1
    mc1 = jnp.where(eqc1, 1.0, 0.0)
    e1 = _idx_dot(w, mc1, 0)
    sc2 = jnp.where(eqc1, s - _MASKV, s)
    c2 = jnp.max(sc2, axis=0, keepdims=True)
    eqc2 = sc2 == c2
    mc2 = jnp.where(eqc2, 1.0, 0.0)
    e2 = _idx_dot(w, mc2, 0)

    sf_ref[2:3, :] = c1
    sf_ref[3:4, :] = c2
    si_ref[2:3, :] = e1[0:1, :].astype(jnp.int32) + q * _BQ
    si_ref[3:4, :] = e2[0:1, :].astype(jnp.int32) + q * _BQ

    bad_x = jnp.max(jnp.where(row_ok, jnp.maximum(d1[1:2, :], d2[1:2, :]), 0.0))
    bad_y = jnp.max(jnp.where(col_ok, jnp.maximum(e1[1:2, :], e2[1:2, :]), 0.0))

    @pl.when(jnp.maximum(bad_x, bad_y) > 1.5)
    def _exact():
        col = jax.lax.broadcasted_iota(jnp.int32, (_BQ, _BK), 1)
        row = jax.lax.broadcasted_iota(jnp.int32, (_BQ, _BK), 0)
        xm1, xa1, xm2, xa2 = _block_top2_lanes(s, col)
        sf_ref[0:1, :] = jnp.transpose(xm1)
        sf_ref[1:2, :] = jnp.transpose(xm2)
        si_ref[0:1, :] = jnp.transpose(xa1) + k * _BK
        si_ref[1:2, :] = jnp.transpose(xa2) + k * _BK
        ym1, yb1, ym2, yb2 = _block_top2_sublanes(s, row)
        sf_ref[2:3, :] = ym1
        sf_ref[3:4, :] = ym2
        si_ref[2:3, :] = yb1 + q * _BQ
        si_ref[3:4, :] = yb2 + q * _BQ

    sl = pl.ds(q * _BQ, _BQ)
    v1, i1 = xv_ref[0:1, sl], xi_ref[0:1, sl]
    v2, i2 = xv_ref[1:2, sl], xi_ref[1:2, sl]
    v1, i1, v2, i2 = _merge_top2(v1, i1, v2, i2, sf_ref[0:1, :], si_ref[0:1, :])
    v1, i1, v2, i2 = _merge_top2(v1, i1, v2, i2, sf_ref[1:2, :], si_ref[1:2, :])
    xv_ref[0:1, sl], xi_ref[0:1, sl] = v1, i1
    xv_ref[1:2, sl], xi_ref[1:2, sl] = v2, i2

    sk = pl.ds(k * _BK, _BK)
    w1, j1 = yv_ref[0:1, sk], yi_ref[0:1, sk]
    w2, j2 = yv_ref[1:2, sk], yi_ref[1:2, sk]
    w1, j1, w2, j2 = _merge_top2(w1, j1, w2, j2, sf_ref[2:3, :], si_ref[2:3, :])
    w1, j1, w2, j2 = _merge_top2(w1, j1, w2, j2, sf_ref[3:4, :], si_ref[3:4, :])
    yv_ref[0:1, sk], yi_ref[0:1, sk] = w1, j1
    yv_ref[1:2, sk], yi_ref[1:2, sk] = w2, j2


def kernel(x_embed, y_embed):
    xn = _normalize_pad(x_embed, _CA, _CB)
    yn = _normalize_pad(y_embed, _CB, _CA)

    xv, xi, yv, yi = pl.pallas_call(
        _topk_kernel,
        grid=(_G, _G),
        in_specs=[pl.BlockSpec((_BQ, _D), lambda q, k: (q, 0)),
                  pl.BlockSpec((_BK, _D), lambda q, k: (k, 0))],
        out_specs=[pl.BlockSpec((2, _NP), lambda q, k: (0, 0)),
                   pl.BlockSpec((2, _NP), lambda q, k: (0, 0)),
                   pl.BlockSpec((2, _NP), lambda q, k: (0, 0)),
                   pl.BlockSpec((2, _NP), lambda q, k: (0, 0))],
        out_shape=[jax.ShapeDtypeStruct((2, _NP), jnp.float32),
                   jax.ShapeDtypeStruct((2, _NP), jnp.int32),
                   jax.ShapeDtypeStruct((2, _NP), jnp.float32),
                   jax.ShapeDtypeStruct((2, _NP), jnp.int32)],
        scratch_shapes=[pltpu.VMEM((4, _BQ), jnp.float32),
                        pltpu.VMEM((4, _BQ), jnp.int32)],
    )(xn, yn)

    return (xv[:, :_N].T, xi[:, :_N].T, yv[:, :_N].T, yi[:, :_N].T)


# speedup vs baseline: 2.5597x; 2.5597x over previous
import jax
import jax.numpy as jnp
from jax.experimental import pallas as pl

_N = 15000
_D0 = 200
_BQ = 512
_BK = 512
_G = 30
_NP = _G * _BQ
_D = 256
_CA = 200
_CB = 201

_NEG = -jnp.inf
_PADV = -1e30
_MASKV = 1e32


def _normalize_pad(a, bias_col_self, bias_col_other):
    an = a / jnp.maximum(jnp.linalg.norm(a, axis=-1, keepdims=True), 1e-8)
    ap = jnp.pad(an, ((0, _NP - _N), (0, _D - _D0)))
    inv = jnp.where(jnp.arange(_NP) >= _N, jnp.float32(_PADV), jnp.float32(0.0))
    ap = ap.at[:, bias_col_self].set(inv)
    ap = ap.at[:, bias_col_other].set(1.0)
    return ap


def _merge_top2(v1, i1, v2, i2, cand_v, cand_i):
    gt1 = cand_v > v1
    gt2 = cand_v > v2
    nv2 = jnp.where(gt1, v1, jnp.where(gt2, cand_v, v2))
    ni2 = jnp.where(gt1, i1, jnp.where(gt2, cand_i, i2))
    nv1 = jnp.where(gt1, cand_v, v1)
    ni1 = jnp.where(gt1, cand_i, i1)
    return nv1, ni1, nv2, ni2


def _weight_rows():
    j = jax.lax.broadcasted_iota(jnp.int32, (1, _BK), 1)
    jsq = j * j
    rows = [
        (j >> 1) << 1,
        j & 1,
        jnp.ones((1, _BK), jnp.int32),
        (jsq >> 12) << 12,
        ((jsq >> 6) & 63) << 6,
        jsq & 63,
    ]
    return jnp.concatenate(rows, axis=0).astype(jnp.float32)


def _mask_stats(w, mask, axis):
    d = jax.lax.dot_general(
        w, mask, (((1,), (axis,)), ((), ())),
        preferred_element_type=jnp.float32,
        precision=jax.lax.Precision.DEFAULT)
    idx_sum = d[0:1, :] + d[1:2, :]
    cnt = d[2:3, :]
    sq_sum = d[3:4, :] + d[4:5, :] + d[5:6, :]
    return idx_sum, cnt, sq_sum


def _resolve_idx(idx_sum, cnt, sq_sum):
    disc = jnp.sqrt(jnp.maximum(2.0 * sq_sum - idx_sum * idx_sum, 0.0))
    amin = jnp.where(cnt < 1.5, idx_sum, (idx_sum - disc) * 0.5)
    apartner = (idx_sum + disc) * 0.5
    return amin, apartner


def _topk_kernel(xn_ref, yn_ref, xv_ref, xi_ref, yv_ref, yi_ref):
    q = pl.program_id(0)
    k = pl.program_id(1)

    @pl.when(jnp.logical_and(q == 0, k == 0))
    def _init():
        xv_ref[...] = jnp.full((2, _NP), _NEG, jnp.float32)
        xi_ref[...] = jnp.zeros((2, _NP), jnp.int32)
        yv_ref[...] = jnp.full((2, _NP), _NEG, jnp.float32)
        yi_ref[...] = jnp.zeros((2, _NP), jnp.int32)

    x = xn_ref[...]
    y = yn_ref[...]
    s = jax.lax.dot_general(
        x, y, (((1,), (1,)), ((), ())),
        preferred_element_type=jnp.float32,
        precision=jax.lax.Precision.DEFAULT)

    w = _weight_rows()

    m1 = jnp.max(s, axis=1, keepdims=True)
    eq1 = s == m1
    mk1 = jnp.where(eq1, 1.0, 0.0)
    s2 = jnp.where(eq1, s - _MASKV, s)
    m2 = jnp.max(s2, axis=1, keepdims=True)
    eq2 = s2 == m2
    mk2 = jnp.where(eq2, 1.0, 0.0)

    sum1, cnt1, sq1 = _mask_stats(w, mk1, 1)
    sum2, cnt2, sq2 = _mask_stats(w, mk2, 1)
    a1, a1b = _resolve_idx(sum1, cnt1, sq1)
    a2, _ = _resolve_idx(sum2, cnt2, sq2)

    m1t = jnp.transpose(m1)
    m2t = jnp.transpose(m2)
    dup1 = cnt1 > 1.5
    cv1, ci1 = m1t, a1
    cv2 = jnp.where(dup1, m1t, m2t)
    ci2 = jnp.where(dup1, a1b, a2)
    ci1 = ci1.astype(jnp.int32) + k * _BK
    ci2 = ci2.astype(jnp.int32) + k * _BK

    c1 = jnp.max(s, axis=0, keepdims=True)
    eqc1 = s == c1
    mc1 = jnp.where(eqc1, 1.0, 0.0)
    sc2 = jnp.where(eqc1, s - _MASKV, s)
    c2 = jnp.max(sc2, axis=0, keepdims=True)
    eqc2 = sc2 == c2
    mc2 = jnp.where(eqc2, 1.0, 0.0)

    sumc1, cntc1, sqc1 = _mask_stats(w, mc1, 0)
    sumc2, cntc2, sqc2 = _mask_stats(w, mc2, 0)
    b1, b1b = _resolve_idx(sumc1, cntc1, sqc1)
    b2, _ = _resolve_idx(sumc2, cntc2, sqc2)

    dupc1 = cntc1 > 1.5
    dv1, di1 = c1, b1
    dv2 = jnp.where(dupc1, c1, c2)
    di2 = jnp.where(dupc1, b1b, b2)
    di1 = di1.astype(jnp.int32) + q * _BQ
    di2 = di2.astype(jnp.int32) + q * _BQ

    sl = pl.ds(q * _BQ, _BQ)
    v1, i1 = xv_ref[0:1, sl], xi_ref[0:1, sl]
    v2, i2 = xv_ref[1:2, sl], xi_ref[1:2, sl]
    v1, i1, v2, i2 = _merge_top2(v1, i1, v2, i2, cv1, ci1)
    v1, i1, v2, i2 = _merge_top2(v1, i1, v2, i2, cv2, ci2)
    xv_ref[0:1, sl], xi_ref[0:1, sl] = v1, i1
    xv_ref[1:2, sl], xi_ref[1:2, sl] = v2, i2

    sk = pl.ds(k * _BK, _BK)
    w1, j1 = yv_ref[0:1, sk], yi_ref[0:1, sk]
    w2, j2 = yv_ref[1:2, sk], yi_ref[1:2, sk]
    w1, j1, w2, j2 = _merge_top2(w1, j1, w2, j2, dv1, di1)
    w1, j1, w2, j2 = _merge_top2(w1, j1, w2, j2, dv2, di2)
    yv_ref[0:1, sk], yi_ref[0:1, sk] = w1, j1
    yv_ref[1:2, sk], yi_ref[1:2, sk] = w2, j2


def kernel(x_embed, y_embed):
    xn = _normalize_pad(x_embed, _CA, _CB)
    yn = _normalize_pad(y_embed, _CB, _CA)

    xv, xi, yv, yi = pl.pallas_call(
        _topk_kernel,
        grid=(_G, _G),
        in_specs=[pl.BlockSpec((_BQ, _D), lambda q, k: (q, 0)),
                  pl.BlockSpec((_BK, _D), lambda q, k: (k, 0))],
        out_specs=[pl.BlockSpec((2, _NP), lambda q, k: (0, 0)),
                   pl.BlockSpec((2, _NP), lambda q, k: (0, 0)),
                   pl.BlockSpec((2, _NP), lambda q, k: (0, 0)),
                   pl.BlockSpec((2, _NP), lambda q, k: (0, 0))],
        out_shape=[jax.ShapeDtypeStruct((2, _NP), jnp.float32),
                   jax.ShapeDtypeStruct((2, _NP), jnp.int32),
                   jax.ShapeDtypeStruct((2, _NP), jnp.float32),
                   jax.ShapeDtypeStruct((2, _NP), jnp.int32)],
    )(xn, yn)

    return (xv[:, :_N].T, xi[:, :_N].T, yv[:, :_N].T, yi[:, :_N].T)


# speedup vs baseline: 3.3137x; 1.2946x over previous
import jax
import jax.numpy as jnp
from jax.experimental import pallas as pl

_N = 15000
_D0 = 200
_BQ = 1024
_BK = 1024
_G = 15
_NP = _G * _BQ
_D = 256
_CA = 200
_CB = 201

_NEG = -jnp.inf
_PADV = -1e30
_MASKV = 1e32


def _normalize_pad(a, bias_col_self, bias_col_other):
    an = a / jnp.maximum(jnp.linalg.norm(a, axis=-1, keepdims=True), 1e-8)
    ap = jnp.pad(an, ((0, _NP - _N), (0, _D - _D0)))
    inv = jnp.where(jnp.arange(_NP) >= _N, jnp.float32(_PADV), jnp.float32(0.0))
    ap = ap.at[:, bias_col_self].set(inv)
    ap = ap.at[:, bias_col_other].set(1.0)
    return ap


def _merge_top2(v1, i1, v2, i2, cand_v, cand_i):
    gt1 = cand_v > v1
    gt2 = cand_v > v2
    nv2 = jnp.where(gt1, v1, jnp.where(gt2, cand_v, v2))
    ni2 = jnp.where(gt1, i1, jnp.where(gt2, cand_i, i2))
    nv1 = jnp.where(gt1, cand_v, v1)
    ni1 = jnp.where(gt1, cand_i, i1)
    return nv1, ni1, nv2, ni2


def _weight_rows():
    j = jax.lax.broadcasted_iota(jnp.int32, (1, _BK), 1)
    jsq = j * j
    rows = [
        (j >> 2) << 2,
        j & 3,
        jnp.ones((1, _BK), jnp.int32),
        (jsq >> 18) << 18,
        ((jsq >> 12) & 63) << 12,
        ((jsq >> 6) & 63) << 6,
        jsq & 63,
    ]
    return jnp.concatenate(rows, axis=0).astype(jnp.float32)


def _mask_stats(w, mask, axis):
    d = jax.lax.dot_general(
        w, mask, (((1,), (axis,)), ((), ())),
        preferred_element_type=jnp.float32,
        precision=jax.lax.Precision.DEFAULT)
    idx_sum = d[0:1, :] + d[1:2, :]
    cnt = d[2:3, :]
    sq_sum = d[3:4, :] + d[4:5, :] + d[5:6, :] + d[6:7, :]
    return idx_sum, cnt, sq_sum


def _resolve_idx(idx_sum, cnt, sq_sum):
    disc = jnp.sqrt(jnp.maximum(2.0 * sq_sum - idx_sum * idx_sum, 0.0))
    amin = jnp.where(cnt < 1.5, idx_sum, (idx_sum - disc) * 0.5)
    apartner = (idx_sum + disc) * 0.5
    return amin, apartner


def _topk_kernel(xn_ref, yn_ref, xv_ref, xi_ref, yv_ref, yi_ref):
    q = pl.program_id(0)
    k = pl.program_id(1)

    @pl.when(jnp.logical_and(q == 0, k == 0))
    def _init():
        xv_ref[...] = jnp.full((2, _NP), _NEG, jnp.float32)
        xi_ref[...] = jnp.zeros((2, _NP), jnp.int32)
        yv_ref[...] = jnp.full((2, _NP), _NEG, jnp.float32)
        yi_ref[...] = jnp.zeros((2, _NP), jnp.int32)

    x = xn_ref[...]
    y = yn_ref[...]
    s = jax.lax.dot_general(
        x, y, (((1,), (1,)), ((), ())),
        preferred_element_type=jnp.float32,
        precision=jax.lax.Precision.DEFAULT)

    w = _weight_rows()

    m1 = jnp.max(s, axis=1, keepdims=True)
    mk1 = jnp.where(s >= m1, 1.0, 0.0)
    s2 = s - mk1 * _MASKV
    m2 = jnp.max(s2, axis=1, keepdims=True)
    mk2 = jnp.where(s2 >= m2, 1.0, 0.0)

    sum1, cnt1, sq1 = _mask_stats(w, mk1, 1)
    sum2, cnt2, sq2 = _mask_stats(w, mk2, 1)
    a1, a1b = _resolve_idx(sum1, cnt1, sq1)
    a2, _ = _resolve_idx(sum2, cnt2, sq2)

    m1t = jnp.transpose(m1)
    m2t = jnp.transpose(m2)
    dup1 = cnt1 > 1.5
    cv1, ci1 = m1t, a1
    cv2 = jnp.where(dup1, m1t, m2t)
    ci2 = jnp.where(dup1, a1b, a2)
    ci1 = ci1.astype(jnp.int32) + k * _BK
    ci2 = ci2.astype(jnp.int32) + k * _BK

    c1 = jnp.max(s, axis=0, keepdims=True)
    mc1 = jnp.where(s >= c1, 1.0, 0.0)
    sc2 = s - mc1 * _MASKV
    c2 = jnp.max(sc2, axis=0, keepdims=True)
    mc2 = jnp.where(sc2 >= c2, 1.0, 0.0)

    sumc1, cntc1, sqc1 = _mask_stats(w, mc1, 0)
    sumc2, cntc2, sqc2 = _mask_stats(w, mc2, 0)
    b1, b1b = _resolve_idx(sumc1, cntc1, sqc1)
    b2, _ = _resolve_idx(sumc2, cntc2, sqc2)

    dupc1 = cntc1 > 1.5
    dv1, di1 = c1, b1
    dv2 = jnp.where(dupc1, c1, c2)
    di2 = jnp.where(dupc1, b1b, b2)
    di1 = di1.astype(jnp.int32) + q * _BQ
    di2 = di2.astype(jnp.int32) + q * _BQ

    sl = pl.ds(q * _BQ, _BQ)
    v1, i1 = xv_ref[0:1, sl], xi_ref[0:1, sl]
    v2, i2 = xv_ref[1:2, sl], xi_ref[1:2, sl]
    v1, i1, v2, i2 = _merge_top2(v1, i1, v2, i2, cv1, ci1)
    v1, i1, v2, i2 = _merge_top2(v1, i1, v2, i2, cv2, ci2)
    xv_ref[0:1, sl], xi_ref[0:1, sl] = v1, i1
    xv_ref[1:2, sl], xi_ref[1:2, sl] = v2, i2

    sk = pl.ds(k * _BK, _BK)
    w1, j1 = yv_ref[0:1, sk], yi_ref[0:1, sk]
    w2, j2 = yv_ref[1:2, sk], yi_ref[1:2, sk]
    w1, j1, w2, j2 = _merge_top2(w1, j1, w2, j2, dv1, di1)
    w1, j1, w2, j2 = _merge_top2(w1, j1, w2, j2, dv2, di2)
    yv_ref[0:1, sk], yi_ref[0:1, sk] = w1, j1
    yv_ref[1:2, sk], yi_ref[1:2, sk] = w2, j2


def kernel(x_embed, y_embed):
    xn = _normalize_pad(x_embed, _CA, _CB)
    yn = _normalize_pad(y_embed, _CB, _CA)

    xv, xi, yv, yi = pl.pallas_call(
        _topk_kernel,
        grid=(_G, _G),
        in_specs=[pl.BlockSpec((_BQ, _D), lambda q, k: (q, 0)),
                  pl.BlockSpec((_BK, _D), lambda q, k: (k, 0))],
        out_specs=[pl.BlockSpec((2, _NP), lambda q, k: (0, 0)),
                   pl.BlockSpec((2, _NP), lambda q, k: (0, 0)),
                   pl.BlockSpec((2, _NP), lambda q, k: (0, 0)),
                   pl.BlockSpec((2, _NP), lambda q, k: (0, 0))],
        out_shape=[jax.ShapeDtypeStruct((2, _NP), jnp.float32),
                   jax.ShapeDtypeStruct((2, _NP), jnp.int32),
                   jax.ShapeDtypeStruct((2, _NP), jnp.float32),
                   jax.ShapeDtypeStruct((2, _NP), jnp.int32)],
    )(xn, yn)

    return (xv[:, :_N].T, xi[:, :_N].T, yv[:, :_N].T, yi[:, :_N].T)
